# baseline (speedup 1.0000x reference)
import dataclasses
import functools

import jax
import jax.numpy as jnp
from jax import lax
from jax.experimental import pallas as pl
from jax.experimental.pallas import tpu as pltpu
from jax.experimental.pallas import tpu_sc as plsc

_P = 3
_Q = 3
_G = 256
_M = 64
_B = 32
_L = 16
_CHUNK = 16

_KSC = 16
_KTC = _B - _KSC
_WPB = 32 // _KSC
_ROWS_W = _G // _WPB
_NCHUNK_W = _ROWS_W // _CHUNK


def _splat(val, dtype=jnp.int32):
    return jnp.full((_L,), val, dtype)


def _sc_body(ctrl_hbm, nu_hbm, nv_hbm, uspan_hbm, vspan_hbm, out_hbm,
             ctrl_v, nu_v, nv_v, uspan_v, vspan_v, tmp_v, ox_v, oy_v, oz_v,
             sem):
    c = lax.axis_index("c")
    s = lax.axis_index("s")
    w = s * 2 + c
    b = w // _WPB
    i0 = (w % _WPB) * _ROWS_W

    pltpu.sync_copy(ctrl_hbm.at[b], ctrl_v)
    pltpu.sync_copy(nu_hbm, nu_v)
    pltpu.sync_copy(nv_hbm, nv_v)
    pltpu.sync_copy(uspan_hbm, uspan_v)
    pltpu.sync_copy(vspan_hbm, vspan_v)

    lane = jax.lax.iota(jnp.int32, _L)
    lane4 = lane * 4
    obufs = (ox_v, oy_v, oz_v)

    def _drain_plane():
        pltpu.make_async_copy(
            ox_v.at[pl.ds(0, _CHUNK)],
            out_hbm.at[b, 0, pl.ds(0, _CHUNK)], sem).wait()

    @pl.loop(0, _NCHUNK_W)
    def _chunk(ic):
        @pl.when(ic >= 2)
        def _():
            _drain_plane()
            _drain_plane()
            _drain_plane()

        @plsc.parallel_loop(0, _CHUNK, unroll=4)
        def _rowa(ii):
            i = i0 + ic * _CHUNK + ii
            i_spl = _splat(0) + i
            u0 = plsc.load_gather(uspan_v, [i_spl]) - _P
            i4_spl = _splat(0) + i * 4
            nur = [plsc.load_gather(nu_v, [i4_spl + r])
                   for r in range(_P + 1)]
            for ng in range(_M // _L):
                for d in range(4):
                    col = lane4 + (ng * 64 + d)
                    acc = nur[0] * plsc.load_gather(ctrl_v, [u0, col])
                    for r in range(1, _P + 1):
                        acc = acc + nur[r] * plsc.load_gather(
                            ctrl_v, [u0 + r, col])
                    tdx = lane + (d * 1024 + ii * 64 + ng * _L)
                    plsc.store_scatter(tmp_v, [tdx], acc)

        @pl.loop(0, _G // _L)
        def _grp(g):
            jdx = lane + g * _L
            vs = plsc.load_gather(vspan_v, [jdx]) - _Q
            j4 = jdx * 4
            idx_s = [vs + s_ for s_ in range(_Q + 1)]
            nvs = [plsc.load_gather(nv_v, [j4 + s_])
                   for s_ in range(_Q + 1)]

            @plsc.parallel_loop(0, _CHUNK, unroll=16)
            def _rowb(ii):
                row = ii * 64
                accs = []
                for d in range(4):
                    base = d * 1024 + row
                    acc = nvs[0] * plsc.load_gather(tmp_v, [idx_s[0] + base])
                    for s_ in range(1, _Q + 1):
                        acc = acc + nvs[s_] * plsc.load_gather(
                            tmp_v, [idx_s[s_] + base])
                    accs.append(acc)
                rw = 1.0 / accs[3]
                row_spl = _splat(0) + (ii + (ic % 2) * _CHUNK)
                col = lane + g * _L
                for d in range(3):
                    plsc.store_scatter(obufs[d], [row_spl, col],
                                       accs[d] * rw)

        for d in range(3):
            pltpu.async_copy(
                obufs[d].at[pl.ds((ic % 2) * _CHUNK, _CHUNK)],
                out_hbm.at[b, d, pl.ds(i0 + ic * _CHUNK, _CHUNK)], sem)

    for _ in range(6):
        _drain_plane()


def _tc_body(uspan_ref, vspan_ref, nu_ref, nvt_ref, ctrl_ref, out_ref,
             bu_ref, bvt_ref):
    t = pl.program_id(0)

    @pl.when(t == 0)
    def _():
        col = jax.lax.broadcasted_iota(jnp.int32, (_G, _M), 1)
        us = uspan_ref[...]
        bu = jnp.zeros((_G, _M), jnp.float32)
        for r in range(_P + 1):
            bu = bu + jnp.where(col == us - _P + r, nu_ref[:, r:r + 1], 0.0)
        bu_ref[...] = bu
        row = jax.lax.broadcasted_iota(jnp.int32, (_M, _G), 0)
        vs = vspan_ref[...]
        bvt = jnp.zeros((_M, _G), jnp.float32)
        for s in range(_Q + 1):
            bvt = bvt + jnp.where(row == vs - _Q + s, nvt_ref[s:s + 1, :], 0.0)
        bvt_ref[...] = bvt

    bu = bu_ref[...]
    bvt = bvt_ref[...]
    planes = []
    for d in range(4):
        a_d = jnp.dot(bu, ctrl_ref[0, d], preferred_element_type=jnp.float32)
        planes.append(jnp.dot(a_d, bvt, preferred_element_type=jnp.float32))
    rw = 1.0 / planes[3]
    for d in range(3):
        out_ref[0, d] = planes[d] * rw


def _tc_eval(ctrl_tc, Nu_uv, Nv_uv, uspan_uv, vspan_uv):
    ctrl_t = ctrl_tc.transpose(0, 3, 1, 2)
    uspan2 = uspan_uv.reshape(_G, 1)
    vspan2 = vspan_uv.reshape(1, _G)
    nvt = Nv_uv.T
    return pl.pallas_call(
        _tc_body,
        grid=(_KTC,),
        in_specs=[
            pl.BlockSpec((_G, 1), lambda t: (0, 0)),
            pl.BlockSpec((1, _G), lambda t: (0, 0)),
            pl.BlockSpec((_G, 4), lambda t: (0, 0)),
            pl.BlockSpec((4, _G), lambda t: (0, 0)),
            pl.BlockSpec((1, 4, _M, _M), lambda t: (t, 0, 0, 0)),
        ],
        out_specs=pl.BlockSpec((1, 3, _G, _G), lambda t: (t, 0, 0, 0)),
        out_shape=jax.ShapeDtypeStruct((_KTC, 3, _G, _G), jnp.float32),
        scratch_shapes=[
            pltpu.VMEM((_G, _M), jnp.float32),
            pltpu.VMEM((_M, _G), jnp.float32),
        ],
    )(uspan2, vspan2, Nu_uv, nvt, ctrl_t)


def kernel(ctrl_pts, Nu_uv, Nv_uv, uspan_uv, vspan_uv):
    mesh = plsc.VectorSubcoreMesh(core_axis_name="c", subcore_axis_name="s")
    cp = pltpu.CompilerParams()
    if "needs_layout_passes" in pltpu.CompilerParams.__dataclass_fields__:
        cp = dataclasses.replace(cp, needs_layout_passes=False)

    @functools.partial(
        pl.kernel,
        mesh=mesh,
        compiler_params=cp,
        out_type=jax.ShapeDtypeStruct((_KSC, 3, _G, _G), jnp.float32),
        scratch_types=[
            pltpu.VMEM((_M, 4 * _M), jnp.float32),
            pltpu.VMEM((4 * _G,), jnp.float32),
            pltpu.VMEM((4 * _G,), jnp.float32),
            pltpu.VMEM((_G,), jnp.int32),
            pltpu.VMEM((_G,), jnp.int32),
            pltpu.VMEM((4 * _CHUNK * _M,), jnp.float32),
            pltpu.VMEM((2 * _CHUNK, _G), jnp.float32),
            pltpu.VMEM((2 * _CHUNK, _G), jnp.float32),
            pltpu.VMEM((2 * _CHUNK, _G), jnp.float32),
            pltpu.SemaphoreType.DMA,
        ],
    )
    def sc_eval(ctrl_hbm, nu_hbm, nv_hbm, uspan_hbm, vspan_hbm, out_hbm,
                ctrl_v, nu_v, nv_v, uspan_v, vspan_v, tmp_v, ox_v, oy_v, oz_v,
                sem):
        _sc_body(ctrl_hbm, nu_hbm, nv_hbm, uspan_hbm, vspan_hbm, out_hbm,
                 ctrl_v, nu_v, nv_v, uspan_v, vspan_v, tmp_v, ox_v, oy_v,
                 oz_v, sem)

    sc_out = sc_eval(ctrl_pts[:_KSC].reshape(_KSC, _M, 4 * _M),
                     Nu_uv.reshape(4 * _G), Nv_uv.reshape(4 * _G),
                     uspan_uv, vspan_uv)
    tc_out = _tc_eval(ctrl_pts[_KSC:], Nu_uv, Nv_uv, uspan_uv, vspan_uv)
    out = jnp.concatenate([sc_out, tc_out], axis=0)
    return out.transpose(0, 2, 3, 1)


# speedup vs baseline: 1.3177x; 1.3177x over previous
import dataclasses
import functools

import jax
import jax.numpy as jnp
from jax import lax
from jax.experimental import pallas as pl
from jax.experimental.pallas import tpu as pltpu
from jax.experimental.pallas import tpu_sc as plsc

_P = 3
_Q = 3
_G = 256
_M = 64
_B = 32
_L = 16
_CHUNK = 16

_KSC = 8
_KTC = _B - _KSC
_WPB = 32 // _KSC
_ROWS_W = _G // _WPB
_NCHUNK_W = _ROWS_W // _CHUNK


def _splat(val, dtype=jnp.int32):
    return jnp.full((_L,), val, dtype)


def _sc_body(ctrl_hbm, nu_hbm, nv_hbm, uspan_hbm, vspan_hbm, out_hbm,
             ctrl_v, nu_v, nv_v, uspan_v, vspan_v, tmp_v, ox_v, oy_v, oz_v,
             sem):
    c = lax.axis_index("c")
    s = lax.axis_index("s")
    w = s * 2 + c
    b = w // _WPB
    i0 = (w % _WPB) * _ROWS_W

    pltpu.sync_copy(ctrl_hbm.at[b], ctrl_v)
    pltpu.sync_copy(nu_hbm, nu_v)
    pltpu.sync_copy(nv_hbm, nv_v)
    pltpu.sync_copy(uspan_hbm, uspan_v)
    pltpu.sync_copy(vspan_hbm, vspan_v)

    lane = jax.lax.iota(jnp.int32, _L)
    lane4 = lane * 4
    obufs = (ox_v, oy_v, oz_v)

    def _drain_plane():
        pltpu.make_async_copy(
            ox_v.at[pl.ds(0, _CHUNK)],
            out_hbm.at[b, 0, pl.ds(0, _CHUNK)], sem).wait()

    @pl.loop(0, _NCHUNK_W)
    def _chunk(ic):
        @pl.when(ic >= 2)
        def _():
            _drain_plane()
            _drain_plane()
            _drain_plane()

        @plsc.parallel_loop(0, _CHUNK, unroll=4)
        def _rowa(ii):
            i = i0 + ic * _CHUNK + ii
            i_spl = _splat(0) + i
            u0 = plsc.load_gather(uspan_v, [i_spl]) - _P
            i4_spl = _splat(0) + i * 4
            nur = [plsc.load_gather(nu_v, [i4_spl + r])
                   for r in range(_P + 1)]
            for ng in range(_M // _L):
                for d in range(4):
                    col = lane4 + (ng * 64 + d)
                    acc = nur[0] * plsc.load_gather(ctrl_v, [u0, col])
                    for r in range(1, _P + 1):
                        acc = acc + nur[r] * plsc.load_gather(
                            ctrl_v, [u0 + r, col])
                    tdx = lane + (d * 1024 + ii * 64 + ng * _L)
                    plsc.store_scatter(tmp_v, [tdx], acc)

        @pl.loop(0, _G // _L)
        def _grp(g):
            jdx = lane + g * _L
            vs = plsc.load_gather(vspan_v, [jdx]) - _Q
            j4 = jdx * 4
            idx_s = [vs + s_ for s_ in range(_Q + 1)]
            nvs = [plsc.load_gather(nv_v, [j4 + s_])
                   for s_ in range(_Q + 1)]

            @plsc.parallel_loop(0, _CHUNK, unroll=16)
            def _rowb(ii):
                row = ii * 64
                accs = []
                for d in range(4):
                    base = d * 1024 + row
                    acc = nvs[0] * plsc.load_gather(tmp_v, [idx_s[0] + base])
                    for s_ in range(1, _Q + 1):
                        acc = acc + nvs[s_] * plsc.load_gather(
                            tmp_v, [idx_s[s_] + base])
                    accs.append(acc)
                rw = 1.0 / accs[3]
                row_spl = _splat(0) + (ii + (ic % 2) * _CHUNK)
                col = lane + g * _L
                for d in range(3):
                    plsc.store_scatter(obufs[d], [row_spl, col],
                                       accs[d] * rw)

        for d in range(3):
            pltpu.async_copy(
                obufs[d].at[pl.ds((ic % 2) * _CHUNK, _CHUNK)],
                out_hbm.at[b, d, pl.ds(i0 + ic * _CHUNK, _CHUNK)], sem)

    for _ in range(6):
        _drain_plane()


def _tc_body(uspan_ref, vspan_ref, nu_ref, nvt_ref, ctrl_ref, out_ref,
             bu_ref, bvt_ref):
    t = pl.program_id(0)

    @pl.when(t == 0)
    def _():
        col = jax.lax.broadcasted_iota(jnp.int32, (_G, _M), 1)
        us = uspan_ref[...]
        bu = jnp.zeros((_G, _M), jnp.float32)
        for r in range(_P + 1):
            bu = bu + jnp.where(col == us - _P + r, nu_ref[:, r:r + 1], 0.0)
        bu_ref[...] = bu
        row = jax.lax.broadcasted_iota(jnp.int32, (_M, _G), 0)
        vs = vspan_ref[...]
        bvt = jnp.zeros((_M, _G), jnp.float32)
        for s in range(_Q + 1):
            bvt = bvt + jnp.where(row == vs - _Q + s, nvt_ref[s:s + 1, :], 0.0)
        bvt_ref[...] = bvt

    bu = bu_ref[...]
    bvt = bvt_ref[...]
    planes = []
    for d in range(4):
        a_d = jnp.dot(bu, ctrl_ref[0, d], preferred_element_type=jnp.float32)
        planes.append(jnp.dot(a_d, bvt, preferred_element_type=jnp.float32))
    rw = 1.0 / planes[3]
    for d in range(3):
        out_ref[0, d] = planes[d] * rw


def _tc_eval(ctrl_tc, Nu_uv, Nv_uv, uspan_uv, vspan_uv):
    ctrl_t = ctrl_tc.transpose(0, 3, 1, 2)
    uspan2 = uspan_uv.reshape(_G, 1)
    vspan2 = vspan_uv.reshape(1, _G)
    nvt = Nv_uv.T
    return pl.pallas_call(
        _tc_body,
        grid=(_KTC,),
        in_specs=[
            pl.BlockSpec((_G, 1), lambda t: (0, 0)),
            pl.BlockSpec((1, _G), lambda t: (0, 0)),
            pl.BlockSpec((_G, 4), lambda t: (0, 0)),
            pl.BlockSpec((4, _G), lambda t: (0, 0)),
            pl.BlockSpec((1, 4, _M, _M), lambda t: (t, 0, 0, 0)),
        ],
        out_specs=pl.BlockSpec((1, 3, _G, _G), lambda t: (t, 0, 0, 0)),
        out_shape=jax.ShapeDtypeStruct((_KTC, 3, _G, _G), jnp.float32),
        scratch_shapes=[
            pltpu.VMEM((_G, _M), jnp.float32),
            pltpu.VMEM((_M, _G), jnp.float32),
        ],
    )(uspan2, vspan2, Nu_uv, nvt, ctrl_t)


def kernel(ctrl_pts, Nu_uv, Nv_uv, uspan_uv, vspan_uv):
    mesh = plsc.VectorSubcoreMesh(core_axis_name="c", subcore_axis_name="s")
    cp = pltpu.CompilerParams()
    if "needs_layout_passes" in pltpu.CompilerParams.__dataclass_fields__:
        cp = dataclasses.replace(cp, needs_layout_passes=False)

    @functools.partial(
        pl.kernel,
        mesh=mesh,
        compiler_params=cp,
        out_type=jax.ShapeDtypeStruct((_KSC, 3, _G, _G), jnp.float32),
        scratch_types=[
            pltpu.VMEM((_M, 4 * _M), jnp.float32),
            pltpu.VMEM((4 * _G,), jnp.float32),
            pltpu.VMEM((4 * _G,), jnp.float32),
            pltpu.VMEM((_G,), jnp.int32),
            pltpu.VMEM((_G,), jnp.int32),
            pltpu.VMEM((4 * _CHUNK * _M,), jnp.float32),
            pltpu.VMEM((2 * _CHUNK, _G), jnp.float32),
            pltpu.VMEM((2 * _CHUNK, _G), jnp.float32),
            pltpu.VMEM((2 * _CHUNK, _G), jnp.float32),
            pltpu.SemaphoreType.DMA,
        ],
    )
    def sc_eval(ctrl_hbm, nu_hbm, nv_hbm, uspan_hbm, vspan_hbm, out_hbm,
                ctrl_v, nu_v, nv_v, uspan_v, vspan_v, tmp_v, ox_v, oy_v, oz_v,
                sem):
        _sc_body(ctrl_hbm, nu_hbm, nv_hbm, uspan_hbm, vspan_hbm, out_hbm,
                 ctrl_v, nu_v, nv_v, uspan_v, vspan_v, tmp_v, ox_v, oy_v,
                 oz_v, sem)

    sc_out = sc_eval(ctrl_pts[:_KSC].reshape(_KSC, _M, 4 * _M),
                     Nu_uv.reshape(4 * _G), Nv_uv.reshape(4 * _G),
                     uspan_uv, vspan_uv)
    tc_out = _tc_eval(ctrl_pts[_KSC:], Nu_uv, Nv_uv, uspan_uv, vspan_uv)
    out = jnp.concatenate([sc_out, tc_out], axis=0)
    return out.transpose(0, 2, 3, 1)
